# baseline (speedup 1.0000x reference)
import jax
import jax.numpy as jnp
from jax import lax
from jax.experimental import pallas as pl
from jax.experimental.pallas import tpu as pltpu
from jax.experimental.pallas import tpu_sc as plsc

BATCH = 4
SEQ = 2048
D_MODEL = 1024

_info = plsc.get_sparse_core_info()
NC, NS = _info.num_cores, _info.num_subcores
NW = NC * NS
POS_PER_W = SEQ // NW
CHUNK = 8
NPC = POS_PER_W // CHUNK
NROUND = NPC * BATCH
NB = 4


def _emb_kernel(tok_hbm, wte_hbm, wpe_hbm, out_hbm,
                idx_v, wpe_v, acc0, acc1, acc2, acc3,
                gsem0, gsem1, gsem2, gsem3,
                ssem0, ssem1, ssem2, ssem3, isem):
    wid = lax.axis_index("s") * NC + lax.axis_index("c")
    pos0 = wid * POS_PER_W
    acc = (acc0, acc1, acc2, acc3)
    gsem = (gsem0, gsem1, gsem2, gsem3)
    ssem = (ssem0, ssem1, ssem2, ssem3)

    idx_descs = [
        pltpu.async_copy(tok_hbm.at[pl.ds(b * SEQ + pos0, POS_PER_W)],
                         idx_v.at[b], isem)
        for b in range(BATCH)
    ]
    wpe_desc = pltpu.async_copy(wpe_hbm.at[pl.ds(pos0, POS_PER_W)], wpe_v,
                                isem)
    for d in idx_descs:
        d.wait()

    def round_coords(r):
        pc = r // BATCH
        b = r - pc * BATCH
        return pc, b

    def gather_desc(r, g):
        pc, b = round_coords(r)
        return pltpu.make_async_copy(
            wte_hbm.at[idx_v.at[b, pl.ds(pc * CHUNK, CHUNK)]],
            acc[g], gsem[g])

    def store_desc(r, g):
        pc, b = round_coords(r)
        return pltpu.make_async_copy(
            acc[g], out_hbm.at[pl.ds(b * SEQ + pos0 + pc * CHUNK, CHUNK)],
            ssem[g])

    for g in range(NB):
        gather_desc(g, g).start()
    wpe_desc.wait()

    @pl.loop(0, NROUND, step=NB)
    def _rounds(r0):
        for g in range(NB):
            r = r0 + g
            gp = (g - 1) % NB

            @pl.when(jnp.logical_and(r >= 1, r - 1 + NB < NROUND))
            def _refill():
                store_desc(r - 1, gp).wait()
                gather_desc(r - 1 + NB, gp).start()

            gather_desc(r, g).wait()
            pc, _ = round_coords(r)
            a = acc[g]
            w0 = pc * CHUNK

            def row_body(row):
                for c in range(D_MODEL // 16):
                    x = wpe_v[w0 + row, pl.ds(c * 16, 16)]
                    plsc.addupdate(a.at[row, pl.ds(c * 16, 16)], x)

            plsc.parallel_loop(0, CHUNK)(row_body)
            store_desc(r, g).start()

    for g in range(NB):
        store_desc(NROUND - NB + g, (NROUND - NB + g) % NB).wait()


@jax.jit
def _run(tok_flat, wte, wpe):
    mesh = plsc.VectorSubcoreMesh(core_axis_name="c", subcore_axis_name="s")
    f = pl.kernel(
        _emb_kernel,
        out_type=jax.ShapeDtypeStruct((BATCH * SEQ, D_MODEL), jnp.float32),
        mesh=mesh,
        scratch_types=[
            pltpu.VMEM((BATCH, POS_PER_W), jnp.int32),
            pltpu.VMEM((POS_PER_W, D_MODEL), jnp.float32),
            pltpu.VMEM((CHUNK, D_MODEL), jnp.float32),
            pltpu.VMEM((CHUNK, D_MODEL), jnp.float32),
            pltpu.VMEM((CHUNK, D_MODEL), jnp.float32),
            pltpu.VMEM((CHUNK, D_MODEL), jnp.float32),
            pltpu.SemaphoreType.DMA,
            pltpu.SemaphoreType.DMA,
            pltpu.SemaphoreType.DMA,
            pltpu.SemaphoreType.DMA,
            pltpu.SemaphoreType.DMA,
            pltpu.SemaphoreType.DMA,
            pltpu.SemaphoreType.DMA,
            pltpu.SemaphoreType.DMA,
            pltpu.SemaphoreType.DMA,
        ],
    )
    return f(tok_flat, wte, wpe)


def kernel(tokens, wte, wpe):
    tok_flat = tokens.reshape(-1).astype(jnp.int32)
    out = _run(tok_flat, wte, wpe)
    return out.reshape(BATCH, SEQ, D_MODEL)


# speedup vs baseline: 1.2224x; 1.2224x over previous
import jax
import jax.numpy as jnp
from jax import lax
from jax.experimental import pallas as pl
from jax.experimental.pallas import tpu as pltpu
from jax.experimental.pallas import tpu_sc as plsc

BATCH = 4
SEQ = 2048
D_MODEL = 1024

_info = plsc.get_sparse_core_info()
NC, NS = _info.num_cores, _info.num_subcores
NW = NC * NS
POS_PER_W = SEQ // NW
CHUNK = 16
NPC = POS_PER_W // CHUNK
NROUND = NPC * BATCH
NB = 3


def _emb_kernel(tok_hbm, wte_hbm, wpe_hbm, out_hbm,
                idx_v, wpe_v, acc0, acc1, acc2,
                gsem0, gsem1, gsem2, ssem0, ssem1, ssem2, isem):
    wid = lax.axis_index("s") * NC + lax.axis_index("c")
    pos0 = wid * POS_PER_W
    acc = (acc0, acc1, acc2)
    gsem = (gsem0, gsem1, gsem2)
    ssem = (ssem0, ssem1, ssem2)

    idx_descs = [
        pltpu.async_copy(tok_hbm.at[pl.ds(b * SEQ + pos0, POS_PER_W)],
                         idx_v.at[b], isem)
        for b in range(BATCH)
    ]
    wpe_desc = pltpu.async_copy(wpe_hbm.at[pl.ds(pos0, POS_PER_W)], wpe_v,
                                isem)
    for d in idx_descs:
        d.wait()

    def gather(r):
        pc, b = divmod(r, BATCH)
        return pltpu.async_copy(
            wte_hbm.at[idx_v.at[b, pl.ds(pc * CHUNK, CHUNK)]],
            acc[r % NB], gsem[r % NB])

    g_descs = {0: gather(0), 1: gather(1)}
    wpe_desc.wait()
    s_descs = {}
    for r in range(NROUND):
        buf = r % NB
        pc, b = divmod(r, BATCH)
        if r + 2 < NROUND:
            if r - 1 in s_descs:
                s_descs[r - 1].wait()
            g_descs[r + 2] = gather(r + 2)
        g_descs[r].wait()
        a = acc[buf]
        w0 = pc * CHUNK

        def col_body(c):
            for row in range(CHUNK):
                x = wpe_v[w0 + row, pl.ds(c, 16)]
                plsc.addupdate(a.at[row, pl.ds(c, 16)], x)

        plsc.parallel_loop(0, D_MODEL, step=16)(col_body)
        s_descs[r] = pltpu.async_copy(
            a, out_hbm.at[pl.ds(b * SEQ + pos0 + pc * CHUNK, CHUNK)],
            ssem[buf])
    for r in range(NROUND - NB, NROUND):
        s_descs[r].wait()


@jax.jit
def _run(tok_flat, wte, wpe):
    mesh = plsc.VectorSubcoreMesh(core_axis_name="c", subcore_axis_name="s")
    f = pl.kernel(
        _emb_kernel,
        out_type=jax.ShapeDtypeStruct((BATCH * SEQ, D_MODEL), jnp.float32),
        mesh=mesh,
        scratch_types=[
            pltpu.VMEM((BATCH, POS_PER_W), jnp.int32),
            pltpu.VMEM((POS_PER_W, D_MODEL), jnp.float32),
            pltpu.VMEM((CHUNK, D_MODEL), jnp.float32),
            pltpu.VMEM((CHUNK, D_MODEL), jnp.float32),
            pltpu.VMEM((CHUNK, D_MODEL), jnp.float32),
            pltpu.SemaphoreType.DMA,
            pltpu.SemaphoreType.DMA,
            pltpu.SemaphoreType.DMA,
            pltpu.SemaphoreType.DMA,
            pltpu.SemaphoreType.DMA,
            pltpu.SemaphoreType.DMA,
            pltpu.SemaphoreType.DMA,
        ],
    )
    return f(tok_flat, wte, wpe)


def kernel(tokens, wte, wpe):
    tok_flat = tokens.reshape(-1).astype(jnp.int32)
    out = _run(tok_flat, wte, wpe)
    return out.reshape(BATCH, SEQ, D_MODEL)


# speedup vs baseline: 1.2938x; 1.0584x over previous
import jax
import jax.numpy as jnp
from jax import lax
from jax.experimental import pallas as pl
from jax.experimental.pallas import tpu as pltpu
from jax.experimental.pallas import tpu_sc as plsc

BATCH = 4
SEQ = 2048
D_MODEL = 1024

_info = plsc.get_sparse_core_info()
NC, NS = _info.num_cores, _info.num_subcores
NW = NC * NS
POS_PER_W = SEQ // NW
CHUNK = 16
NPC = POS_PER_W // CHUNK
NROUND = NPC * BATCH
NB = 3


def _emb_kernel(tok_hbm, wte_hbm, wpe_hbm, out_hbm,
                idx_v, wpe_v, acc0, acc1, acc2,
                gsem0, gsem1, gsem2, ssem0, ssem1, ssem2, isem):
    wid = lax.axis_index("s") * NC + lax.axis_index("c")
    pos0 = wid * POS_PER_W
    acc = (acc0, acc1, acc2)
    gsem = (gsem0, gsem1, gsem2)
    ssem = (ssem0, ssem1, ssem2)

    idx_descs = [
        pltpu.async_copy(tok_hbm.at[pl.ds(b * SEQ + pos0, POS_PER_W)],
                         idx_v.at[b], isem)
        for b in range(BATCH)
    ]
    wpe_desc = pltpu.async_copy(wpe_hbm.at[pl.ds(pos0, POS_PER_W)], wpe_v,
                                isem)
    for d in idx_descs:
        d.wait()

    def gather(r):
        pc, b = divmod(r, BATCH)
        return pltpu.async_copy(
            wte_hbm.at[idx_v.at[b, pl.ds(pc * CHUNK, CHUNK)]],
            acc[r % NB], gsem[r % NB])

    g_descs = {r: gather(r) for r in range(NB - 1)}
    wpe_desc.wait()
    s_descs = {}
    for r in range(NROUND):
        buf = r % NB
        pc, b = divmod(r, BATCH)
        g_descs[r].wait()
        a = acc[buf]
        w0 = pc * CHUNK

        def col_body(c):
            for row in range(CHUNK):
                x = wpe_v[w0 + row, pl.ds(c, 16)]
                plsc.addupdate(a.at[row, pl.ds(c, 16)], x)

        plsc.parallel_loop(0, D_MODEL, step=16)(col_body)
        s_descs[r] = pltpu.async_copy(
            a, out_hbm.at[pl.ds(b * SEQ + pos0 + pc * CHUNK, CHUNK)],
            ssem[buf])
        if r + NB - 1 < NROUND:
            if r - 1 in s_descs:
                s_descs[r - 1].wait()
            g_descs[r + NB - 1] = gather(r + NB - 1)
    for r in range(NROUND - NB, NROUND):
        s_descs[r].wait()


@jax.jit
def _run(tok_flat, wte, wpe):
    mesh = plsc.VectorSubcoreMesh(core_axis_name="c", subcore_axis_name="s")
    f = pl.kernel(
        _emb_kernel,
        out_type=jax.ShapeDtypeStruct((BATCH * SEQ, D_MODEL), jnp.float32),
        mesh=mesh,
        scratch_types=[
            pltpu.VMEM((BATCH, POS_PER_W), jnp.int32),
            pltpu.VMEM((POS_PER_W, D_MODEL), jnp.float32),
            pltpu.VMEM((CHUNK, D_MODEL), jnp.float32),
            pltpu.VMEM((CHUNK, D_MODEL), jnp.float32),
            pltpu.VMEM((CHUNK, D_MODEL), jnp.float32),
            pltpu.SemaphoreType.DMA,
            pltpu.SemaphoreType.DMA,
            pltpu.SemaphoreType.DMA,
            pltpu.SemaphoreType.DMA,
            pltpu.SemaphoreType.DMA,
            pltpu.SemaphoreType.DMA,
            pltpu.SemaphoreType.DMA,
        ],
    )
    return f(tok_flat, wte, wpe)


def kernel(tokens, wte, wpe):
    tok_flat = tokens.reshape(-1).astype(jnp.int32)
    out = _run(tok_flat, wte, wpe)
    return out.reshape(BATCH, SEQ, D_MODEL)


# speedup vs baseline: 1.3180x; 1.0187x over previous
import jax
import jax.numpy as jnp
from jax import lax
from jax.experimental import pallas as pl
from jax.experimental.pallas import tpu as pltpu
from jax.experimental.pallas import tpu_sc as plsc

BATCH = 4
SEQ = 2048
D_MODEL = 1024

_info = plsc.get_sparse_core_info()
NC, NS = _info.num_cores, _info.num_subcores
NW = NC * NS
POS_PER_W = SEQ // NW
CHUNK = 16
NPC = POS_PER_W // CHUNK
NROUND = NPC * BATCH
NB = 3


def _emb_kernel(tok_hbm, wte_hbm, wpe_hbm, out_hbm,
                idx_v, wpe_v, acc0, acc1, acc2,
                gsem0, gsem1, gsem2, ssem0, ssem1, ssem2, isem, wsem):
    wid = lax.axis_index("s") * NC + lax.axis_index("c")
    pos0 = wid * POS_PER_W
    acc = (acc0, acc1, acc2)
    gsem = (gsem0, gsem1, gsem2)
    ssem = (ssem0, ssem1, ssem2)

    idx_descs = [
        pltpu.async_copy(tok_hbm.at[pl.ds(b * SEQ + pos0, POS_PER_W)],
                         idx_v.at[b], isem)
        for b in range(BATCH)
    ]
    wpe_descs = [
        pltpu.async_copy(wpe_hbm.at[pl.ds(pos0 + pc * CHUNK, CHUNK)],
                         wpe_v.at[pl.ds(pc * CHUNK, CHUNK)], wsem)
        for pc in range(NPC)
    ]
    for d in idx_descs:
        d.wait()

    def gather(r):
        pc, b = divmod(r, BATCH)
        return pltpu.async_copy(
            wte_hbm.at[idx_v.at[b, pl.ds(pc * CHUNK, CHUNK)]],
            acc[r % NB], gsem[r % NB])

    g_descs = {r: gather(r) for r in range(NB - 1)}
    s_descs = {}
    for r in range(NROUND):
        buf = r % NB
        pc, b = divmod(r, BATCH)
        if b == 0:
            wpe_descs[pc].wait()
        g_descs[r].wait()
        a = acc[buf]
        w0 = pc * CHUNK

        def col_body(c):
            for row in range(CHUNK):
                x = wpe_v[w0 + row, pl.ds(c, 16)]
                plsc.addupdate(a.at[row, pl.ds(c, 16)], x)

        plsc.parallel_loop(0, D_MODEL, step=16)(col_body)
        s_descs[r] = pltpu.async_copy(
            a, out_hbm.at[pl.ds(b * SEQ + pos0 + pc * CHUNK, CHUNK)],
            ssem[buf])
        if r + NB - 1 < NROUND:
            if r - 1 in s_descs:
                s_descs[r - 1].wait()
            g_descs[r + NB - 1] = gather(r + NB - 1)
    for r in range(NROUND - NB, NROUND):
        s_descs[r].wait()


@jax.jit
def _run(tok_flat, wte, wpe):
    mesh = plsc.VectorSubcoreMesh(core_axis_name="c", subcore_axis_name="s")
    f = pl.kernel(
        _emb_kernel,
        out_type=jax.ShapeDtypeStruct((BATCH * SEQ, D_MODEL), jnp.float32),
        mesh=mesh,
        scratch_types=[
            pltpu.VMEM((BATCH, POS_PER_W), jnp.int32),
            pltpu.VMEM((POS_PER_W, D_MODEL), jnp.float32),
            pltpu.VMEM((CHUNK, D_MODEL), jnp.float32),
            pltpu.VMEM((CHUNK, D_MODEL), jnp.float32),
            pltpu.VMEM((CHUNK, D_MODEL), jnp.float32),
            pltpu.SemaphoreType.DMA,
            pltpu.SemaphoreType.DMA,
            pltpu.SemaphoreType.DMA,
            pltpu.SemaphoreType.DMA,
            pltpu.SemaphoreType.DMA,
            pltpu.SemaphoreType.DMA,
            pltpu.SemaphoreType.DMA,
            pltpu.SemaphoreType.DMA,
        ],
    )
    return f(tok_flat, wte, wpe)


def kernel(tokens, wte, wpe):
    tok_flat = tokens.reshape(-1).astype(jnp.int32)
    out = _run(tok_flat, wte, wpe)
    return out.reshape(BATCH, SEQ, D_MODEL)
